# baseline (speedup 1.0000x reference)
import jax
import jax.numpy as jnp
from jax import lax
from jax.experimental import pallas as pl
from jax.experimental.pallas import tpu as pltpu
from jax.experimental.pallas import tpu_sc as plsc

N_NODES = 10000
N_EDGES = 320000
D_FEAT = 128
DH = 64
CHUNK = 128
NS = 16
NC = 2
NPAD = 10240
BLK = 80
NBLK = NPAD // BLK
NOUT = N_NODES // BLK
GSZ = 8
NBUF = 4
GROUPS = 20
CPT = GSZ * GROUPS
NCHUNKS = CPT * NS
E_PAD = NCHUNKS * CHUNK


def _gin_body(node, srcp, dst3, eps, out, tbl, acc, bufb, rows,
              sidx2, didx2, epsv, gs0, gs1, gs2, gs3, ss0, ss1, ss2, ss3,
              is0, is1):
    c = lax.axis_index("c")
    s = lax.axis_index("s")
    gsems = (gs0, gs1, gs2, gs3)
    ssems = (ss0, ss1, ss2, ss3)
    isems = (is0, is1)
    col0 = pl.multiple_of(c * DH, DH)

    def idx_load(j0, par):
        pltpu.async_copy(srcp.at[pl.ds(j0, GSZ)], sidx2.at[par], isems[par])
        pltpu.async_copy(dst3.at[pl.ds(j0, GSZ)], didx2.at[par], isems[par])

    def idx_wait(par):
        pltpu.make_async_copy(srcp.at[pl.ds(0, GSZ)], sidx2.at[par],
                              isems[par]).wait()
        pltpu.make_async_copy(dst3.at[pl.ds(0, GSZ)], didx2.at[par],
                              isems[par]).wait()

    idx_load(s * CPT, 0)

    pltpu.sync_copy(eps, epsv)
    scale = epsv[...] + 1.0

    def srow(r, carry):
        for c4 in range(DH // 16):
            dsl = pl.ds(c4 * 16, 16)
            bufb[r, dsl] = bufb[r, dsl] * scale
        return carry

    def stage(k, carry):
        blk = s + NS * k

        @pl.when(blk < NOUT)
        def _():
            off = blk * BLK
            pltpu.sync_copy(node.at[pl.ds(off, BLK), pl.ds(col0, DH)], bufb)
            pltpu.sync_copy(bufb, tbl.at[pl.ds(off, BLK)])
            lax.fori_loop(0, BLK, srow, 0)
            pltpu.sync_copy(bufb, acc.at[pl.ds(off, BLK)])

        return carry

    lax.fori_loop(0, NBLK // NS, stage, 0)
    plsc.subcore_barrier()

    def wait_scatter(slot, par):
        pltpu.make_async_copy(rows.at[slot], acc.at[didx2.at[par, 0]],
                              ssems[slot]).wait()

    def pair(gg, carry):
        for par in range(2):
            g = gg * 2 + par
            idx_wait(par)
            jn = jnp.minimum(s * CPT + (g + 1) * GSZ, NCHUNKS - GSZ)
            sidx8 = sidx2.at[par]
            didx8 = didx2.at[par]
            gcp = {}
            for q in range(GSZ):
                slot = q % NBUF
                if q < NBUF and par == 0:
                    @pl.when(gg >= 1)
                    def _():
                        wait_scatter(slot, par)
                else:
                    wait_scatter(slot, par)
                if q == NBUF:
                    idx_load(jn, 1 - par)
                gcp[q] = pltpu.async_copy(
                    tbl.at[sidx8.at[q]], rows.at[slot], gsems[slot])
                if q >= 1:
                    p = q - 1
                    gcp[p].wait()
                    pltpu.async_copy(
                        rows.at[p % NBUF], acc.at[didx8.at[p]],
                        ssems[p % NBUF], add=True)
            gcp[GSZ - 1].wait()
            pltpu.async_copy(
                rows.at[(GSZ - 1) % NBUF], acc.at[didx8.at[GSZ - 1]],
                ssems[(GSZ - 1) % NBUF], add=True)
        return carry

    lax.fori_loop(0, GROUPS // 2, pair, 0)
    for slot in range(NBUF):
        wait_scatter(slot, 0)
    idx_wait(0)
    plsc.subcore_barrier()

    def fout(k, carry):
        blk = s + NS * k

        @pl.when(blk < NOUT)
        def _():
            off = blk * BLK
            pltpu.sync_copy(acc.at[pl.ds(off, BLK)],
                            out.at[pl.ds(off, BLK), pl.ds(col0, DH)])

        return carry

    lax.fori_loop(0, NBLK // NS, fout, 0)


def kernel(node, edge_index, eps_k):
    epad = E_PAD - N_EDGES
    srcp = jnp.concatenate(
        [edge_index[1],
         jnp.full((epad,), N_NODES, jnp.int32)]).reshape(NCHUNKS, CHUNK)
    dst3 = jnp.concatenate(
        [edge_index[0],
         jnp.full((epad,), NPAD - 1, jnp.int32)]).reshape(NCHUNKS, CHUNK)
    eps = jnp.broadcast_to(jnp.reshape(eps_k.astype(jnp.float32), (1,)), (16,))

    mesh = plsc.VectorSubcoreMesh(core_axis_name="c", subcore_axis_name="s")
    run = pl.kernel(
        _gin_body,
        out_type=jax.ShapeDtypeStruct((N_NODES, D_FEAT), jnp.float32),
        mesh=mesh,
        compiler_params=pltpu.CompilerParams(use_tc_tiling_on_sc=False),
        scratch_types=[
            pltpu.VMEM_SHARED((NPAD, DH), jnp.float32),
            pltpu.VMEM_SHARED((NPAD, DH), jnp.float32),
            pltpu.VMEM((BLK, DH), jnp.float32),
            pltpu.VMEM((NBUF, CHUNK, DH), jnp.float32),
            pltpu.VMEM((2, GSZ, CHUNK), jnp.int32),
            pltpu.VMEM((2, GSZ, CHUNK), jnp.int32),
            pltpu.VMEM((16,), jnp.float32),
        ] + [pltpu.SemaphoreType.DMA] * 10,
    )
    return run(node, srcp, dst3, eps)


# speedup vs baseline: 1.0370x; 1.0370x over previous
import jax
import jax.numpy as jnp
from jax import lax
from jax.experimental import pallas as pl
from jax.experimental.pallas import tpu as pltpu
from jax.experimental.pallas import tpu_sc as plsc

N_NODES = 10000
N_EDGES = 320000
D_FEAT = 128
DH = 64
CHUNK = 128
NS = 16
NC = 2
NPAD = 10240
BLK = 80
NBLK = NPAD // BLK
NOUT = N_NODES // BLK
GSZ = 8
NBUF = 4
GROUPS = 20
CPT = GSZ * GROUPS
NCHUNKS = CPT * NS
E_PAD = NCHUNKS * CHUNK


def _gin_body(node, srcp, dst3, eps, out, tbl, acc, rows,
              sidx2, didx2, epsv, gs0, gs1, gs2, gs3, ss0, ss1, ss2, ss3,
              is0, is1):
    c = lax.axis_index("c")
    s = lax.axis_index("s")
    gsems = (gs0, gs1, gs2, gs3)
    ssems = (ss0, ss1, ss2, ss3)
    isems = (is0, is1)
    col0 = pl.multiple_of(c * DH, DH)

    def idx_load(j0, par):
        pltpu.async_copy(srcp.at[pl.ds(j0, GSZ)], sidx2.at[par], isems[par])
        pltpu.async_copy(dst3.at[pl.ds(j0, GSZ)], didx2.at[par], isems[par])

    def idx_wait(par):
        pltpu.make_async_copy(srcp.at[pl.ds(0, GSZ)], sidx2.at[par],
                              isems[par]).wait()
        pltpu.make_async_copy(dst3.at[pl.ds(0, GSZ)], didx2.at[par],
                              isems[par]).wait()

    idx_load(s * CPT, 0)

    pltpu.sync_copy(eps, epsv)
    scale = epsv[...] + 1.0

    KPT = NBLK // NS

    def srow(slot):
        def body(r, carry):
            for c4 in range(DH // 16):
                dsl = pl.ds(c4 * 16, 16)
                rows[slot, r, dsl] = rows[slot, r, dsl] * scale
            return carry
        return body

    def stage_read(k, slot):
        blk = s + NS * k
        off = blk * BLK

        @pl.when(blk < NOUT)
        def _():
            pltpu.async_copy(node.at[pl.ds(off, BLK), pl.ds(col0, DH)],
                             rows.at[slot, pl.ds(0, BLK)], gsems[slot])

    def stage_wait(k, slot):
        @pl.when(s + NS * k < NOUT)
        def _():
            pltpu.make_async_copy(node.at[pl.ds(0, BLK), pl.ds(col0, DH)],
                                  rows.at[slot, pl.ds(0, BLK)],
                                  gsems[slot]).wait()

    stage_read(0, 0)
    for k in range(KPT):
        slot = k % 2
        blk = s + NS * k
        off = blk * BLK
        stage_wait(k, slot)
        if k + 1 < KPT:
            stage_read(k + 1, 1 - slot)

        @pl.when(blk < NOUT)
        def _():
            pltpu.sync_copy(rows.at[slot, pl.ds(0, BLK)],
                            tbl.at[pl.ds(off, BLK)])
            lax.fori_loop(0, BLK, srow(slot), 0)
            pltpu.sync_copy(rows.at[slot, pl.ds(0, BLK)],
                            acc.at[pl.ds(off, BLK)])

    plsc.subcore_barrier()

    def wait_scatter(slot, par):
        pltpu.make_async_copy(rows.at[slot], acc.at[didx2.at[par, 0]],
                              ssems[slot]).wait()

    def pair(gg, carry):
        for par in range(2):
            g = gg * 2 + par
            idx_wait(par)
            jn = jnp.minimum(s * CPT + (g + 1) * GSZ, NCHUNKS - GSZ)
            sidx8 = sidx2.at[par]
            didx8 = didx2.at[par]
            gcp = {}
            for q in range(GSZ):
                slot = q % NBUF
                if q < NBUF and par == 0:
                    @pl.when(gg >= 1)
                    def _():
                        wait_scatter(slot, par)
                else:
                    wait_scatter(slot, par)
                if q == NBUF:
                    idx_load(jn, 1 - par)
                gcp[q] = pltpu.async_copy(
                    tbl.at[sidx8.at[q]], rows.at[slot], gsems[slot])
                if q >= 1:
                    p = q - 1
                    gcp[p].wait()
                    pltpu.async_copy(
                        rows.at[p % NBUF], acc.at[didx8.at[p]],
                        ssems[p % NBUF], add=True)
            gcp[GSZ - 1].wait()
            pltpu.async_copy(
                rows.at[(GSZ - 1) % NBUF], acc.at[didx8.at[GSZ - 1]],
                ssems[(GSZ - 1) % NBUF], add=True)
        return carry

    lax.fori_loop(0, GROUPS // 2, pair, 0)
    for slot in range(NBUF):
        wait_scatter(slot, 0)
    idx_wait(0)
    plsc.subcore_barrier()

    for k in range(NBLK // NS):
        blk = s + NS * k
        off = blk * BLK

        @pl.when(blk < NOUT)
        def _():
            pltpu.async_copy(acc.at[pl.ds(off, BLK)],
                             out.at[pl.ds(off, BLK), pl.ds(col0, DH)],
                             ssems[k % NBUF])

    for k in range(NBLK // NS):
        blk = s + NS * k
        off = blk * BLK

        @pl.when(blk < NOUT)
        def _():
            pltpu.make_async_copy(acc.at[pl.ds(off, BLK)],
                                  out.at[pl.ds(off, BLK), pl.ds(col0, DH)],
                                  ssems[k % NBUF]).wait()


def kernel(node, edge_index, eps_k):
    epad = E_PAD - N_EDGES
    srcp = jnp.concatenate(
        [edge_index[1],
         jnp.full((epad,), N_NODES, jnp.int32)]).reshape(NCHUNKS, CHUNK)
    dst3 = jnp.concatenate(
        [edge_index[0],
         jnp.full((epad,), NPAD - 1, jnp.int32)]).reshape(NCHUNKS, CHUNK)
    eps = jnp.broadcast_to(jnp.reshape(eps_k.astype(jnp.float32), (1,)), (16,))

    mesh = plsc.VectorSubcoreMesh(core_axis_name="c", subcore_axis_name="s")
    run = pl.kernel(
        _gin_body,
        out_type=jax.ShapeDtypeStruct((N_NODES, D_FEAT), jnp.float32),
        mesh=mesh,
        compiler_params=pltpu.CompilerParams(use_tc_tiling_on_sc=False),
        scratch_types=[
            pltpu.VMEM_SHARED((NPAD, DH), jnp.float32),
            pltpu.VMEM_SHARED((NPAD, DH), jnp.float32),
            pltpu.VMEM((NBUF, CHUNK, DH), jnp.float32),
            pltpu.VMEM((2, GSZ, CHUNK), jnp.int32),
            pltpu.VMEM((2, GSZ, CHUNK), jnp.int32),
            pltpu.VMEM((16,), jnp.float32),
        ] + [pltpu.SemaphoreType.DMA] * 10,
    )
    return run(node, srcp, dst3, eps)


# speedup vs baseline: 1.1402x; 1.0996x over previous
import jax
import jax.numpy as jnp
from jax import lax
from jax.experimental import pallas as pl
from jax.experimental.pallas import tpu as pltpu
from jax.experimental.pallas import tpu_sc as plsc

N_NODES = 10000
N_EDGES = 320000
D_FEAT = 128
DH = 64
CHUNK = 128
NS = 16
NC = 2
NPAD = 10240
BLK = 80
NBLK = NPAD // BLK
NOUT = N_NODES // BLK
GSZ = 16
NBUF = 4
GROUPS = 10
CPT = GSZ * GROUPS
NCHUNKS = CPT * NS
E_PAD = NCHUNKS * CHUNK


def _gin_body(node, idx2, eps, out, tbl, acc, rows,
              pidx2, epsv, gs0, gs1, gs2, gs3, ss0, ss1, ss2, ss3,
              is0, is1):
    c = lax.axis_index("c")
    s = lax.axis_index("s")
    gsems = (gs0, gs1, gs2, gs3)
    ssems = (ss0, ss1, ss2, ss3)
    isems = (is0, is1)
    col0 = pl.multiple_of(c * DH, DH)

    def idx_load(j0, par):
        pltpu.async_copy(idx2.at[pl.ds(j0, GSZ)], pidx2.at[par], isems[par])

    def idx_wait(par):
        pltpu.make_async_copy(idx2.at[pl.ds(0, GSZ)], pidx2.at[par],
                              isems[par]).wait()

    idx_load(s * CPT, 0)

    pltpu.sync_copy(eps, epsv)
    scale = epsv[...] + 1.0

    KPT = NBLK // NS

    def srow(slot):
        def body(r, carry):
            for c4 in range(DH // 16):
                dsl = pl.ds(c4 * 16, 16)
                rows[slot, r, dsl] = rows[slot, r, dsl] * scale
            return carry
        return body

    def stage_read(k, slot):
        blk = s + NS * k
        off = blk * BLK

        @pl.when(blk < NOUT)
        def _():
            pltpu.async_copy(node.at[pl.ds(off, BLK), pl.ds(col0, DH)],
                             rows.at[slot, pl.ds(0, BLK)], gsems[slot])

    def stage_wait(k, slot):
        @pl.when(s + NS * k < NOUT)
        def _():
            pltpu.make_async_copy(node.at[pl.ds(0, BLK), pl.ds(col0, DH)],
                                  rows.at[slot, pl.ds(0, BLK)],
                                  gsems[slot]).wait()

    stage_read(0, 0)
    for k in range(KPT):
        slot = k % 2
        blk = s + NS * k
        off = blk * BLK
        stage_wait(k, slot)
        if k + 1 < KPT:
            stage_read(k + 1, 1 - slot)

        @pl.when(blk < NOUT)
        def _():
            pltpu.sync_copy(rows.at[slot, pl.ds(0, BLK)],
                            tbl.at[pl.ds(off, BLK)])
            lax.fori_loop(0, BLK, srow(slot), 0)
            pltpu.sync_copy(rows.at[slot, pl.ds(0, BLK)],
                            acc.at[pl.ds(off, BLK)])

    plsc.subcore_barrier()

    def wait_scatter(slot, par):
        pltpu.make_async_copy(rows.at[slot], acc.at[pidx2.at[0, 0, 1]],
                              ssems[slot]).wait()

    def pair(gg, carry):
        for par in range(2):
            g = gg * 2 + par
            idx_wait(par)
            jn = jnp.minimum(s * CPT + (g + 1) * GSZ, NCHUNKS - GSZ)
            gcp = {}
            for q in range(GSZ):
                slot = q % NBUF
                if q < NBUF and par == 0:
                    @pl.when(gg >= 1)
                    def _():
                        wait_scatter(slot, par)
                else:
                    wait_scatter(slot, par)
                if q == NBUF:
                    idx_load(jn, 1 - par)
                gcp[q] = pltpu.async_copy(
                    tbl.at[pidx2.at[par, q, 0]], rows.at[slot], gsems[slot])
                if q >= 1:
                    p = q - 1
                    gcp[p].wait()
                    pltpu.async_copy(
                        rows.at[p % NBUF], acc.at[pidx2.at[par, p, 1]],
                        ssems[p % NBUF], add=True)
            gcp[GSZ - 1].wait()
            pltpu.async_copy(
                rows.at[(GSZ - 1) % NBUF], acc.at[pidx2.at[par, GSZ - 1, 1]],
                ssems[(GSZ - 1) % NBUF], add=True)
        return carry

    lax.fori_loop(0, GROUPS // 2, pair, 0)
    for slot in range(NBUF):
        wait_scatter(slot, 0)
    idx_wait(0)
    plsc.subcore_barrier()

    for k in range(NBLK // NS):
        blk = s + NS * k
        off = blk * BLK

        @pl.when(blk < NOUT)
        def _():
            pltpu.async_copy(acc.at[pl.ds(off, BLK)],
                             out.at[pl.ds(off, BLK), pl.ds(col0, DH)],
                             ssems[k % NBUF])

    for k in range(NBLK // NS):
        blk = s + NS * k
        off = blk * BLK

        @pl.when(blk < NOUT)
        def _():
            pltpu.make_async_copy(acc.at[pl.ds(off, BLK)],
                                  out.at[pl.ds(off, BLK), pl.ds(col0, DH)],
                                  ssems[k % NBUF]).wait()


def kernel(node, edge_index, eps_k):
    epad = E_PAD - N_EDGES
    srcp = jnp.concatenate(
        [edge_index[1],
         jnp.full((epad,), N_NODES, jnp.int32)]).reshape(NCHUNKS, CHUNK)
    dst3 = jnp.concatenate(
        [edge_index[0],
         jnp.full((epad,), NPAD - 1, jnp.int32)]).reshape(NCHUNKS, CHUNK)
    idx2 = jnp.stack([srcp, dst3], axis=1)
    eps = jnp.broadcast_to(jnp.reshape(eps_k.astype(jnp.float32), (1,)), (16,))

    mesh = plsc.VectorSubcoreMesh(core_axis_name="c", subcore_axis_name="s")
    run = pl.kernel(
        _gin_body,
        out_type=jax.ShapeDtypeStruct((N_NODES, D_FEAT), jnp.float32),
        mesh=mesh,
        compiler_params=pltpu.CompilerParams(use_tc_tiling_on_sc=False),
        scratch_types=[
            pltpu.VMEM_SHARED((NPAD, DH), jnp.float32),
            pltpu.VMEM_SHARED((NPAD, DH), jnp.float32),
            pltpu.VMEM((NBUF, CHUNK, DH), jnp.float32),
            pltpu.VMEM((2, GSZ, 2, CHUNK), jnp.int32),
            pltpu.VMEM((16,), jnp.float32),
        ] + [pltpu.SemaphoreType.DMA] * 10,
    )
    return run(node, idx2, eps)
